# baseline (speedup 1.0000x reference)
import functools

import jax
import jax.numpy as jnp
from jax import lax
from jax.experimental import pallas as pl
from jax.experimental.pallas import tpu as pltpu
from jax.experimental.pallas import tpu_sc as plsc

_N = 512
_B = 16
_TR = 64
_NR = _N // _TR

_K = 2
_ROWS = _K * _N
_NW = 32
_RPW = _ROWS // _NW
_CH = 16
_NCH = _RPW // _CH

_ROT = ((13, 15, 26, 6), (17, 29, 16, 24))
_KS = (0x0, 0x2A, 0x1BD11BDA ^ 0x0 ^ 0x2A)


def _threefry_bits(ctr):
    x0 = jnp.zeros_like(ctr) + jnp.uint32(_KS[0])
    x1 = ctr + jnp.uint32(_KS[1])
    for i in range(5):
        for r in _ROT[i % 2]:
            x0 = x0 + x1
            x1 = (x1 << jnp.uint32(r)) | (x1 >> jnp.uint32(32 - r))
            x1 = x1 ^ x0
        x0 = x0 + jnp.uint32(_KS[(i + 1) % 3])
        x1 = x1 + jnp.uint32(_KS[(i + 2) % 3] + i + 1)
    return x0 ^ x1


def _bits_to_uniform(bits):
    fb = (bits >> jnp.uint32(9)) | jnp.uint32(0x3F800000)
    f = jax.lax.bitcast_convert_type(fb, jnp.float32) - jnp.float32(1.0)
    minv = jnp.float32(1e-6)
    span = jnp.float32((1.0 - 1e-6) - 1e-6)
    return jnp.maximum(minv, f * span + minv)


_mesh = plsc.VectorSubcoreMesh(
    core_axis_name="c", subcore_axis_name="s", num_cores=2, num_subcores=16
)


@functools.partial(
    pl.kernel,
    out_type=jax.ShapeDtypeStruct((_ROWS, _N), jnp.float32),
    mesh=_mesh,
    scratch_types=[
        pltpu.VMEM((_B + 16,), jnp.int32),
        pltpu.VMEM((_CH, _N), jnp.float32),
        pltpu.VMEM((_CH, _N), jnp.float32),
    ],
)
def _sc_sample(a_hbm, len_hbm, out_hbm, len_v, a_v, o_v):
    wid = lax.axis_index("c") * 16 + lax.axis_index("s")
    row0 = wid * _RPW
    pltpu.sync_copy(len_hbm, len_v.at[pl.ds(0, _B)])
    lane = lax.iota(jnp.int32, 16)

    @pl.loop(0, _NCH)
    def _chunk(ci):
        r_start = row0 + ci * _CH
        pltpu.sync_copy(a_hbm.at[pl.ds(r_start, _CH)], a_v)

        @pl.loop(0, _CH)
        def _row(r):
            grow = r_start + r
            rib = grow & (_N - 1)
            batch = lax.shift_right_logical(grow, 9)
            ln_s = len_v[pl.ds(batch, 16)][0]
            lim = ln_s & lax.shift_right_arithmetic(rib - ln_s, 31)
            limv = jnp.full((16,), lim, jnp.int32)
            rv = jnp.full((16,), rib, jnp.int32)
            ctr_base = grow * _N

            @pl.loop(0, _N // 16)
            def _col(j):
                col = lane + j * 16
                ctr = (ctr_base + col).astype(jnp.uint32)
                u = _bits_to_uniform(_threefry_bits(ctr))
                a = a_v[r, pl.ds(j * 16, 16)]
                y = u / (u + (jnp.float32(1.0) - u) * jnp.exp(-a))
                zero = jnp.float32(0.0)
                y = jnp.where(col != rv, y, zero)
                y = jnp.where(col < limv, y, zero)
                o_v[r, pl.ds(j * 16, 16)] = y

        pltpu.sync_copy(o_v, out_hbm.at[pl.ds(r_start, _CH)])


def _tc_body(len_ref, a_ref, o_ref):
    b = pl.program_id(0) + _K
    ln = len_ref[b]
    nr = jax.lax.div(ln + (_TR - 1), _TR)

    @pl.loop(0, nr)
    def _live_rows(ri):
        r0 = ri * _TR
        rows = r0 + jax.lax.broadcasted_iota(jnp.int32, (_TR, _N), 0)
        cols = jax.lax.broadcasted_iota(jnp.int32, (_TR, _N), 1)
        a = a_ref[0, pl.ds(r0, _TR), :]
        ctr = (b * (_N * _N) + rows * _N + cols).astype(jnp.uint32)
        u = _bits_to_uniform(_threefry_bits(ctr))
        y = u / (u + (jnp.float32(1.0) - u) * jnp.exp(-a))
        m = (rows < ln) & (cols < ln) & (rows != cols)
        o_ref[0, pl.ds(r0, _TR), :] = jnp.where(m, y, jnp.float32(0.0))

    @pl.loop(nr, _NR)
    def _dead_rows(ri):
        o_ref[0, pl.ds(ri * _TR, _TR), :] = jnp.zeros((_TR, _N), jnp.float32)


def kernel(A, lengths):
    lengths32 = lengths.astype(jnp.int32)
    sc_out = _sc_sample(A.reshape(_B * _N, _N), lengths32)
    tc_out = pl.pallas_call(
        _tc_body,
        grid=(_B - _K,),
        in_specs=[
            pl.BlockSpec(memory_space=pltpu.SMEM),
            pl.BlockSpec((1, _N, _N), lambda b: (b + _K, 0, 0)),
        ],
        out_specs=pl.BlockSpec((1, _N, _N), lambda b: (b + _K, 0, 0)),
        out_shape=jax.ShapeDtypeStruct((_B, _N, _N), jnp.float32),
    )(lengths32, A)
    return lax.dynamic_update_slice(
        tc_out, sc_out.reshape(_K, _N, _N), (0, 0, 0)
    )


# speedup vs baseline: 1.0045x; 1.0045x over previous
import functools

import jax
import jax.numpy as jnp
from jax import lax
from jax.experimental import pallas as pl
from jax.experimental.pallas import tpu as pltpu
from jax.experimental.pallas import tpu_sc as plsc

_N = 512
_B = 16
_TR = 64
_NR = _N // _TR

_K = 2
_ROWS = _K * _N
_NW = 32
_RPW = _ROWS // _NW
_CH = 16
_NCH = _RPW // _CH

_ROT = ((13, 15, 26, 6), (17, 29, 16, 24))
_KS = (0x0, 0x2A, 0x1BD11BDA ^ 0x0 ^ 0x2A)


def _threefry_bits(ctr):
    x0 = jnp.zeros_like(ctr) + jnp.uint32(_KS[0])
    x1 = ctr + jnp.uint32(_KS[1])
    for i in range(5):
        for r in _ROT[i % 2]:
            x0 = x0 + x1
            x1 = (x1 << jnp.uint32(r)) | (x1 >> jnp.uint32(32 - r))
            x1 = x1 ^ x0
        x0 = x0 + jnp.uint32(_KS[(i + 1) % 3])
        x1 = x1 + jnp.uint32(_KS[(i + 2) % 3] + i + 1)
    return x0 ^ x1


def _bits_to_uniform(bits):
    fb = (bits >> jnp.uint32(9)) | jnp.uint32(0x3F800000)
    f = jax.lax.bitcast_convert_type(fb, jnp.float32) - jnp.float32(1.0)
    minv = jnp.float32(1e-6)
    span = jnp.float32((1.0 - 1e-6) - 1e-6)
    return jnp.maximum(minv, f * span + minv)


_mesh = plsc.VectorSubcoreMesh(
    core_axis_name="c", subcore_axis_name="s", num_cores=2, num_subcores=16
)


@functools.partial(
    pl.kernel,
    out_type=jax.ShapeDtypeStruct((_ROWS, _N), jnp.float32),
    mesh=_mesh,
    scratch_types=[
        pltpu.VMEM((_B + 16,), jnp.int32),
        pltpu.VMEM((_CH, _N), jnp.float32),
        pltpu.VMEM((_CH, _N), jnp.float32),
    ],
)
def _sc_sample(a_hbm, len_hbm, out_hbm, len_v, a_v, o_v):
    wid = lax.axis_index("c") * 16 + lax.axis_index("s")
    row0 = wid * _RPW
    pltpu.sync_copy(len_hbm, len_v.at[pl.ds(0, _B)])
    lane = lax.iota(jnp.int32, 16)

    @pl.loop(0, _NCH)
    def _chunk(ci):
        r_start = row0 + ci * _CH
        pltpu.sync_copy(a_hbm.at[pl.ds(r_start, _CH)], a_v)

        @pl.loop(0, _CH)
        def _row(r):
            grow = r_start + r
            rib = grow & (_N - 1)
            batch = lax.shift_right_logical(grow, 9)
            ln_s = len_v[pl.ds(batch, 16)][0]
            lim = ln_s & lax.shift_right_arithmetic(rib - ln_s, 31)
            limv = jnp.full((16,), lim, jnp.int32)
            rv = jnp.full((16,), rib, jnp.int32)
            ctr_base = grow * _N

            @pl.loop(0, _N // 16)
            def _col(j):
                col = lane + j * 16
                ctr = (ctr_base + col).astype(jnp.uint32)
                u = _bits_to_uniform(_threefry_bits(ctr))
                a = a_v[r, pl.ds(j * 16, 16)]
                y = u / (u + (jnp.float32(1.0) - u) * jnp.exp(-a))
                zero = jnp.float32(0.0)
                y = jnp.where(col != rv, y, zero)
                y = jnp.where(col < limv, y, zero)
                o_v[r, pl.ds(j * 16, 16)] = y

        pltpu.sync_copy(o_v, out_hbm.at[pl.ds(r_start, _CH)])


def _tc_body(len_ref, a_ref, o_ref):
    b = pl.program_id(0) + _K
    ln = len_ref[b]
    nr = jax.lax.div(ln + (_TR - 1), _TR)

    @pl.loop(0, nr)
    def _live_rows(ri):
        r0 = ri * _TR
        rows = r0 + jax.lax.broadcasted_iota(jnp.int32, (_TR, _N), 0)
        cols = jax.lax.broadcasted_iota(jnp.int32, (_TR, _N), 1)
        a = a_ref[0, pl.ds(r0, _TR), :]
        ctr = (b * (_N * _N) + rows * _N + cols).astype(jnp.uint32)
        u = _bits_to_uniform(_threefry_bits(ctr))
        y = u / (u + (jnp.float32(1.0) - u) * jnp.exp(-a))
        m = (rows < ln) & (cols < ln) & (rows != cols)
        o_ref[0, pl.ds(r0, _TR), :] = jnp.where(m, y, jnp.float32(0.0))

    @pl.loop(nr, _NR)
    def _dead_rows(ri):
        o_ref[0, pl.ds(ri * _TR, _TR), :] = jnp.zeros((_TR, _N), jnp.float32)


def _merge_body(s_ref, t_ref, o_ref):
    del t_ref
    o_ref[0] = s_ref[0]


def kernel(A, lengths):
    lengths32 = lengths.astype(jnp.int32)
    sc_out = _sc_sample(A.reshape(_B * _N, _N), lengths32)
    tc_out = pl.pallas_call(
        _tc_body,
        grid=(_B - _K,),
        in_specs=[
            pl.BlockSpec(memory_space=pltpu.SMEM),
            pl.BlockSpec((1, _N, _N), lambda b: (b + _K, 0, 0)),
        ],
        out_specs=pl.BlockSpec((1, _N, _N), lambda b: (b + _K, 0, 0)),
        out_shape=jax.ShapeDtypeStruct((_B, _N, _N), jnp.float32),
    )(lengths32, A)
    return pl.pallas_call(
        _merge_body,
        grid=(_K,),
        in_specs=[
            pl.BlockSpec((1, _N, _N), lambda b: (b, 0, 0)),
            pl.BlockSpec(memory_space=pl.ANY),
        ],
        out_specs=pl.BlockSpec((1, _N, _N), lambda b: (b, 0, 0)),
        out_shape=jax.ShapeDtypeStruct((_B, _N, _N), jnp.float32),
        input_output_aliases={1: 0},
    )(sc_out.reshape(_K, _N, _N), tc_out)


# speedup vs baseline: 1.3027x; 1.2969x over previous
import jax
import jax.numpy as jnp
from jax.experimental import pallas as pl
from jax.experimental.pallas import tpu as pltpu

_N = 512
_B = 16
_TR = 64
_NR = _N // _TR

_ROT = ((13, 15, 26, 6), (17, 29, 16, 24))
_KS = (0x0, 0x2A, 0x1BD11BDA ^ 0x0 ^ 0x2A)


def _threefry_bits(ctr):
    x0 = jnp.zeros_like(ctr) + jnp.uint32(_KS[0])
    x1 = ctr + jnp.uint32(_KS[1])
    for i in range(5):
        for r in _ROT[i % 2]:
            x0 = x0 + x1
            x1 = (x1 << jnp.uint32(r)) | (x1 >> jnp.uint32(32 - r))
            x1 = x1 ^ x0
        x0 = x0 + jnp.uint32(_KS[(i + 1) % 3])
        x1 = x1 + jnp.uint32(_KS[(i + 2) % 3] + i + 1)
    return x0 ^ x1


def _bits_to_uniform(bits):
    fb = (bits >> jnp.uint32(9)) | jnp.uint32(0x3F800000)
    f = jax.lax.bitcast_convert_type(fb, jnp.float32) - jnp.float32(1.0)
    minv = jnp.float32(1e-6)
    span = jnp.float32((1.0 - 1e-6) - 1e-6)
    return jnp.maximum(minv, f * span + minv)


def _body(len_ref, a_ref, o_ref):
    b = pl.program_id(0)
    ln = len_ref[b]
    nr = jax.lax.div(ln + (_TR - 1), _TR)
    nr2 = jax.lax.div(nr, 2)

    def _live_tile(r0, tr):
        rows = r0 + jax.lax.broadcasted_iota(jnp.int32, (tr, _N), 0)
        cols = jax.lax.broadcasted_iota(jnp.int32, (tr, _N), 1)
        a = a_ref[0, pl.ds(r0, tr), :]
        ctr = (b * (_N * _N) + rows * _N + cols).astype(jnp.uint32)
        u = _bits_to_uniform(_threefry_bits(ctr))
        y = u / (u + (jnp.float32(1.0) - u) * jnp.exp(-a))
        m = (rows < ln) & (cols < ln) & (rows != cols)
        o_ref[0, pl.ds(r0, tr), :] = jnp.where(m, y, jnp.float32(0.0))

    @pl.loop(0, nr2)
    def _live_pairs(ri):
        _live_tile(ri * (2 * _TR), 2 * _TR)

    @pl.loop(2 * nr2, nr)
    def _live_tail(ri):
        _live_tile(ri * _TR, _TR)

    @pl.loop(nr, _NR)
    def _dead_rows(ri):
        o_ref[0, pl.ds(ri * _TR, _TR), :] = jnp.zeros((_TR, _N), jnp.float32)


def kernel(A, lengths):
    lengths32 = lengths.astype(jnp.int32)
    return pl.pallas_call(
        _body,
        grid=(_B,),
        in_specs=[
            pl.BlockSpec(memory_space=pltpu.SMEM),
            pl.BlockSpec((1, _N, _N), lambda b: (b, 0, 0)),
        ],
        out_specs=pl.BlockSpec((1, _N, _N), lambda b: (b, 0, 0)),
        out_shape=jax.ShapeDtypeStruct((_B, _N, _N), jnp.float32),
    )(lengths32, A)
